# baseline (speedup 1.0000x reference)
import functools

import jax
import jax.numpy as jnp
from jax import lax
from jax.experimental import pallas as pl
from jax.experimental.pallas import tpu as pltpu
from jax.experimental.pallas import tpu_sc as plsc
from jax._src.pallas import mpmd as _plmpmd

_THRES = 0.5
_L = 4096
_BR = 256
_NB = _L // _BR
_NW = 32
_IDX_TOTAL = 3 * _L
_PER_W = _IDX_TOTAL // _NW
_CHUNK = 128


def _rowstat_body(x_ref, base_ref, idx1_ref, idx2_ref, idxd_ref):
    pi = pl.program_id(0)
    x = x_ref[...]
    col = jax.lax.broadcasted_iota(jnp.int32, (_BR, _L), 1)
    g = pi * _BR + jax.lax.broadcasted_iota(jnp.int32, (_BR, 1), 0)
    masked = jnp.where(col == g, -jnp.inf, x)
    vmax = jnp.max(masked, axis=1, keepdims=True)
    cand = jnp.where(masked == vmax, col, _L)
    inds = jnp.min(cand, axis=1, keepdims=True)
    m = vmax > _THRES
    diag = g * (_L + 1)
    idx1 = jnp.where(m, g * _L + inds, diag)
    idx2 = jnp.where(m, inds * _L + g, diag)
    base_ref[...] = jnp.zeros((_BR, _L), jnp.float32)
    idx1_ref[...] = idx1[None]
    idx2_ref[...] = idx2[None]
    idxd_ref[...] = diag[None]


_sc_mesh = plsc.VectorSubcoreMesh(core_axis_name="c", subcore_axis_name="s")


def _sc_scatter_body(idx_hbm, base_hbm, out_hbm, idx_v, ones_v, sem):
    del base_hbm
    wid = lax.axis_index("s") * 2 + lax.axis_index("c")
    for t in range(_CHUNK // 16):
        ones_v[pl.ds(t * 16, 16)] = jnp.full((16,), 1.0, jnp.float32)
    base = wid * _PER_W
    for k in range(_PER_W // _CHUNK):
        pltpu.sync_copy(idx_hbm.at[pl.ds(base + k * _CHUNK, _CHUNK)], idx_v)
        pltpu.async_copy(ones_v, out_hbm.at[idx_v], sem).wait()


_sc_scatter = _plmpmd._mpmd_map(
    [(_sc_mesh, _sc_scatter_body)],
    out_types=[jax.ShapeDtypeStruct((_L * _L,), jnp.float32)],
    input_output_aliases={1: 0},
    scratch_types=[
        pltpu.VMEM((_CHUNK,), jnp.int32),
        pltpu.VMEM((_CHUNK,), jnp.float32),
        pltpu.SemaphoreType.DMA,
    ],
)


def kernel(input):
    x = input.reshape(_L, _L)

    idx_spec = pl.BlockSpec((1, _BR, 1), lambda i: (i, 0, 0))
    idx_shape = jax.ShapeDtypeStruct((_NB, _BR, 1), jnp.int32)
    base, idx1, idx2, idxd = pl.pallas_call(
        _rowstat_body,
        grid=(_NB,),
        in_specs=[pl.BlockSpec((_BR, _L), lambda i: (i, 0))],
        out_specs=[
            pl.BlockSpec((_BR, _L), lambda i: (i, 0)),
            idx_spec,
            idx_spec,
            idx_spec,
        ],
        out_shape=[
            jax.ShapeDtypeStruct((_L, _L), jnp.float32),
            idx_shape,
            idx_shape,
            idx_shape,
        ],
    )(x)

    idx_all = jnp.concatenate(
        [idx1.reshape(_L), idx2.reshape(_L), idxd.reshape(_L)]
    )

    (out,) = _sc_scatter(idx_all, base.reshape(_L * _L))
    return out.reshape(input.shape)


# speedup vs baseline: 3.1168x; 3.1168x over previous
import jax
import jax.numpy as jnp
from jax.experimental import pallas as pl

_THRES = 0.5
_L = 4096
_BR = 256
_NB = _L // _BR


def _rowstat_body(x_ref, a_ref):
    pi = pl.program_id(0)
    x = x_ref[...]
    col = jax.lax.broadcasted_iota(jnp.int32, (_BR, _L), 1)
    g = pi * _BR + jax.lax.broadcasted_iota(jnp.int32, (_BR, 1), 0)
    masked = jnp.where(col == g, -jnp.inf, x)
    vmax = jnp.max(masked, axis=1, keepdims=True)
    cand = jnp.where(masked == vmax, col, _L)
    inds = jnp.min(cand, axis=1, keepdims=True)
    a_ref[...] = jnp.where(vmax > _THRES, inds, -1)


def _assemble_body(a_c_ref, a_r_ref, out_ref):
    pi = pl.program_id(0)
    rowi = jax.lax.broadcasted_iota(jnp.int32, (_BR, _L), 0)
    coli = jax.lax.broadcasted_iota(jnp.int32, (_BR, _L), 1)
    g = rowi + pi * _BR
    a_i = a_c_ref[pl.ds(pi * _BR, _BR), :]
    a_j = a_r_ref[...]
    hit = (coli == g) | (coli == a_i) | (a_j == g)
    out_ref[...] = hit.astype(jnp.float32)


def kernel(input):
    x = input.reshape(_L, _L)

    a_c = pl.pallas_call(
        _rowstat_body,
        grid=(_NB,),
        in_specs=[pl.BlockSpec((_BR, _L), lambda i: (i, 0))],
        out_specs=pl.BlockSpec((_BR, 1), lambda i: (i, 0)),
        out_shape=jax.ShapeDtypeStruct((_L, 1), jnp.int32),
    )(x)

    a_r = a_c.reshape(1, _L)

    out2d = pl.pallas_call(
        _assemble_body,
        grid=(_NB,),
        in_specs=[
            pl.BlockSpec((_L, 1), lambda i: (0, 0)),
            pl.BlockSpec((1, _L), lambda i: (0, 0)),
        ],
        out_specs=pl.BlockSpec((_BR, _L), lambda i: (i, 0)),
        out_shape=jax.ShapeDtypeStruct((_L, _L), jnp.float32),
    )(a_c, a_r)

    return out2d.reshape(input.shape)


# speedup vs baseline: 3.3974x; 1.0900x over previous
import jax
import jax.numpy as jnp
from jax.experimental import pallas as pl

_THRES = 0.5
_L = 4096
_BR = 512
_NB = _L // _BR


def _rowstat_body(x_ref, a_ref):
    pi = pl.program_id(0)
    x = x_ref[...]
    col = jax.lax.broadcasted_iota(jnp.int32, (_BR, _L), 1)
    g = pi * _BR + jax.lax.broadcasted_iota(jnp.int32, (_BR, 1), 0)
    masked = jnp.where(col == g, -jnp.inf, x)
    vmax = jnp.max(masked, axis=1, keepdims=True)
    cand = jnp.where(masked == vmax, col, _L)
    inds = jnp.min(cand, axis=1, keepdims=True)
    a_ref[...] = jnp.where(vmax > _THRES, inds, -1)


def _assemble_body(a_c_ref, a_r_ref, out_ref):
    pi = pl.program_id(0)
    rowi = jax.lax.broadcasted_iota(jnp.int32, (_BR, _L), 0)
    coli = jax.lax.broadcasted_iota(jnp.int32, (_BR, _L), 1)
    g = rowi + pi * _BR
    a_i = a_c_ref[pl.ds(pi * _BR, _BR), :]
    a_j = a_r_ref[...]
    hit = (coli == g) | (coli == a_i) | (a_j == g)
    out_ref[...] = hit.astype(jnp.float32)


def kernel(input):
    x = input.reshape(_L, _L)

    a_c = pl.pallas_call(
        _rowstat_body,
        grid=(_NB,),
        in_specs=[pl.BlockSpec((_BR, _L), lambda i: (i, 0))],
        out_specs=pl.BlockSpec((_BR, 1), lambda i: (i, 0)),
        out_shape=jax.ShapeDtypeStruct((_L, 1), jnp.int32),
    )(x)

    a_r = a_c.reshape(1, _L)

    out2d = pl.pallas_call(
        _assemble_body,
        grid=(_NB,),
        in_specs=[
            pl.BlockSpec((_L, 1), lambda i: (0, 0)),
            pl.BlockSpec((1, _L), lambda i: (0, 0)),
        ],
        out_specs=pl.BlockSpec((_BR, _L), lambda i: (i, 0)),
        out_shape=jax.ShapeDtypeStruct((_L, _L), jnp.float32),
    )(a_c, a_r)

    return out2d.reshape(input.shape)
